# baseline (speedup 1.0000x reference)
import functools

import jax
import jax.numpy as jnp
from jax import lax
from jax.experimental import pallas as pl
from jax.experimental.pallas import tpu as pltpu
from jax.experimental.pallas import tpu_sc as plsc

BATCH = 16384
FACTOR = 32
L = 16
NUM_WORKERS = 32
B_PER_W = BATCH // NUM_WORKERS
CHUNK = 128
N_CHUNKS = B_PER_W // CHUNK
GROUPS = B_PER_W // L
import numpy as np

HI_MASK = np.int32(-65536)


def _decode(w):
    lo = lax.bitcast_convert_type(lax.shift_left(w, 16), jnp.float32)
    hi = lax.bitcast_convert_type(lax.bitwise_and(w, HI_MASK), jnp.float32)
    return lo, hi


def _bpr_kernel(user, item_i, item_j, user_emb, item_emb,
                out_i, out_j,
                idx_u, idx_i, idx_j,
                rows_u, rows_i, rows_j,
                acc_i_v, acc_j_v, sem):
    c = lax.axis_index("c")
    s = lax.axis_index("s")
    wid = s * 2 + c
    base = wid * B_PER_W

    pltpu.sync_copy(user.at[pl.ds(base, B_PER_W)], idx_u)
    pltpu.sync_copy(item_i.at[pl.ds(base, B_PER_W)], idx_i)
    pltpu.sync_copy(item_j.at[pl.ds(base, B_PER_W)], idx_j)

    copies = []
    for cch in range(N_CHUNKS):
        sl = pl.ds(cch * CHUNK, CHUNK)
        copies.append(pltpu.make_async_copy(
            user_emb.at[idx_u.at[sl]], rows_u.at[sl], sem))
        copies.append(pltpu.make_async_copy(
            item_emb.at[idx_i.at[sl]], rows_i.at[sl], sem))
        copies.append(pltpu.make_async_copy(
            item_emb.at[idx_j.at[sl]], rows_j.at[sl], sem))
    for cp in copies:
        cp.start()
    for cp in copies:
        cp.wait()

    lanes = lax.iota(jnp.int32, L)

    def group_body(g, carry):
        row0 = g * L
        acc_i = jnp.zeros((L,), jnp.float32)
        acc_j = jnp.zeros((L,), jnp.float32)
        for l in range(L):
            b = row0 + l
            wu = plsc.bitcast(rows_u[b, :], jnp.int32)
            wi = plsc.bitcast(rows_i[b, :], jnp.int32)
            wj = plsc.bitcast(rows_j[b, :], jnp.int32)
            u0, u1 = _decode(wu)
            vi0, vi1 = _decode(wi)
            vj0, vj1 = _decode(wj)
            di = jnp.sum(u0 * vi0 + u1 * vi1)
            dj = jnp.sum(u0 * vj0 + u1 * vj1)
            mask = lanes == l
            acc_i = jnp.where(mask, di, acc_i)
            acc_j = jnp.where(mask, dj, acc_j)
        acc_i_v[pl.ds(row0, L)] = acc_i
        acc_j_v[pl.ds(row0, L)] = acc_j
        return carry

    lax.fori_loop(0, GROUPS, group_body, 0)

    pltpu.sync_copy(acc_i_v, out_i.at[pl.ds(base, B_PER_W)])
    pltpu.sync_copy(acc_j_v, out_j.at[pl.ds(base, B_PER_W)])


@jax.jit
def kernel(user, item_i, item_j, user_emb, item_emb):
    mesh = plsc.VectorSubcoreMesh(core_axis_name="c", subcore_axis_name="s")
    f32 = jnp.float32
    bf16 = jnp.bfloat16
    run = functools.partial(
        pl.kernel,
        out_type=(jax.ShapeDtypeStruct((BATCH,), f32),
                  jax.ShapeDtypeStruct((BATCH,), f32)),
        mesh=mesh,
        scratch_types=[
            pltpu.VMEM((B_PER_W,), jnp.int32),
            pltpu.VMEM((B_PER_W,), jnp.int32),
            pltpu.VMEM((B_PER_W,), jnp.int32),
            pltpu.VMEM((B_PER_W, FACTOR), bf16),
            pltpu.VMEM((B_PER_W, FACTOR), bf16),
            pltpu.VMEM((B_PER_W, FACTOR), bf16),
            pltpu.VMEM((B_PER_W,), f32),
            pltpu.VMEM((B_PER_W,), f32),
            pltpu.SemaphoreType.DMA,
        ],
        compiler_params=pltpu.CompilerParams(
            needs_layout_passes=False, use_tc_tiling_on_sc=False),
    )(_bpr_kernel)
    return run(user.astype(jnp.int32), item_i.astype(jnp.int32),
               item_j.astype(jnp.int32),
               user_emb.astype(bf16), item_emb.astype(bf16))


# speedup vs baseline: 1.1729x; 1.1729x over previous
import functools

import jax
import jax.numpy as jnp
from jax import lax
from jax.experimental import pallas as pl
from jax.experimental.pallas import tpu as pltpu
from jax.experimental.pallas import tpu_sc as plsc

BATCH = 16384
FACTOR = 32
L = 16
NUM_WORKERS = 32
B_PER_W = BATCH // NUM_WORKERS
CHUNK = 128
N_CHUNKS = B_PER_W // CHUNK
GROUPS = B_PER_W // L


def _bpr_kernel(user, item_i, item_j, user_emb, item_emb,
                out_i, out_j,
                idx_u, idx_i, idx_j,
                rows_u, rows_i, rows_j,
                acc_i_v, acc_j_v, sem):
    c = lax.axis_index("c")
    s = lax.axis_index("s")
    wid = s * 2 + c
    base = wid * B_PER_W

    pltpu.sync_copy(user.at[pl.ds(base, B_PER_W)], idx_u)
    pltpu.sync_copy(item_i.at[pl.ds(base, B_PER_W)], idx_i)
    pltpu.sync_copy(item_j.at[pl.ds(base, B_PER_W)], idx_j)

    copies = []
    for cch in range(N_CHUNKS):
        sl = pl.ds(cch * CHUNK, CHUNK)
        copies.append(pltpu.make_async_copy(
            user_emb.at[idx_u.at[sl]], rows_u.at[sl], sem))
        copies.append(pltpu.make_async_copy(
            item_emb.at[idx_i.at[sl]], rows_i.at[sl], sem))
        copies.append(pltpu.make_async_copy(
            item_emb.at[idx_j.at[sl]], rows_j.at[sl], sem))
    for cp in copies:
        cp.start()
    for cp in copies:
        cp.wait()

    lanes = lax.iota(jnp.int32, L)

    def group_body(g, carry):
        row0 = g * L
        acc_i = jnp.zeros((L,), jnp.float32)
        acc_j = jnp.zeros((L,), jnp.float32)
        for l in range(L):
            b = row0 + l
            u0 = rows_u[b, pl.ds(0, L)]
            u1 = rows_u[b, pl.ds(L, L)]
            vi0 = rows_i[b, pl.ds(0, L)]
            vi1 = rows_i[b, pl.ds(L, L)]
            vj0 = rows_j[b, pl.ds(0, L)]
            vj1 = rows_j[b, pl.ds(L, L)]
            di = jnp.sum(u0 * vi0 + u1 * vi1)
            dj = jnp.sum(u0 * vj0 + u1 * vj1)
            mask = lanes == l
            acc_i = jnp.where(mask, di, acc_i)
            acc_j = jnp.where(mask, dj, acc_j)
        acc_i_v[pl.ds(row0, L)] = acc_i
        acc_j_v[pl.ds(row0, L)] = acc_j
        return carry

    lax.fori_loop(0, GROUPS, group_body, 0)

    pltpu.sync_copy(acc_i_v, out_i.at[pl.ds(base, B_PER_W)])
    pltpu.sync_copy(acc_j_v, out_j.at[pl.ds(base, B_PER_W)])


@jax.jit
def kernel(user, item_i, item_j, user_emb, item_emb):
    mesh = plsc.VectorSubcoreMesh(core_axis_name="c", subcore_axis_name="s")
    f32 = jnp.float32
    run = functools.partial(
        pl.kernel,
        out_type=(jax.ShapeDtypeStruct((BATCH,), f32),
                  jax.ShapeDtypeStruct((BATCH,), f32)),
        mesh=mesh,
        scratch_types=[
            pltpu.VMEM((B_PER_W,), jnp.int32),
            pltpu.VMEM((B_PER_W,), jnp.int32),
            pltpu.VMEM((B_PER_W,), jnp.int32),
            pltpu.VMEM((B_PER_W, FACTOR), f32),
            pltpu.VMEM((B_PER_W, FACTOR), f32),
            pltpu.VMEM((B_PER_W, FACTOR), f32),
            pltpu.VMEM((B_PER_W,), f32),
            pltpu.VMEM((B_PER_W,), f32),
            pltpu.SemaphoreType.DMA,
        ],
        compiler_params=pltpu.CompilerParams(
            needs_layout_passes=False, use_tc_tiling_on_sc=False),
    )(_bpr_kernel)
    return run(user.astype(jnp.int32), item_i.astype(jnp.int32),
               item_j.astype(jnp.int32), user_emb, item_emb)
